# baseline (speedup 1.0000x reference)
import functools

import jax
import jax.numpy as jnp
import numpy as np
from jax import lax
from jax.experimental import pallas as pl
from jax.experimental.pallas import tpu as pltpu
from jax.experimental.pallas import tpu_sc as plsc

_NC = 2
_NS = 16
_NW = _NC * _NS
_LANES = 16
_CHUNK = 16


def _pos_encoding(max_len, d_model):
    pos = np.arange(max_len, dtype=np.float32)[:, None]
    i = np.arange(0, d_model, 2, dtype=np.float32)
    div = np.power(10000.0, i / d_model)
    pe = np.zeros((max_len, d_model), dtype=np.float32)
    pe[:, 0::2] = np.sin(pos / div)
    pe[:, 1::2] = np.cos(pos / div)
    return pe


@functools.partial(jax.jit, static_argnums=(3, 4))
def _embed(W, idx_flat, pe, seq, d_model):
    n = idx_flat.shape[0]
    per_w = n // _NW
    nsub = per_w // _CHUNK
    vregs_per_row = d_model // _LANES

    mesh = plsc.VectorSubcoreMesh(core_axis_name="c", subcore_axis_name="s")

    @functools.partial(
        pl.kernel,
        out_type=jax.ShapeDtypeStruct((n, d_model), jnp.float32),
        mesh=mesh,
        scratch_types=[
            pltpu.VMEM((per_w,), jnp.int32),
            pltpu.VMEM((_CHUNK, d_model), jnp.float32),
            pltpu.VMEM((_CHUNK, d_model), jnp.float32),
            pltpu.SemaphoreType.DMA,
            pltpu.SemaphoreType.DMA,
        ],
    )
    def k(w_hbm, idx_hbm, pe_hbm, out_hbm, idx_v, rows_v, pe_v, gsem, psem):
        wid = lax.axis_index("s") * _NC + lax.axis_index("c")
        tok0 = wid * per_w
        pos0 = lax.rem(tok0, seq)

        pltpu.sync_copy(idx_hbm.at[pl.ds(tok0, per_w)], idx_v)

        def sub(j, carry):
            base = j * _CHUNK
            gcp = pltpu.async_copy(
                w_hbm.at[idx_v.at[pl.ds(base, _CHUNK)]], rows_v, gsem
            )
            pcp = pltpu.async_copy(pe_hbm.at[pl.ds(pos0 + base, _CHUNK)], pe_v, psem)
            gcp.wait()
            pcp.wait()

            def add_row(r, c2):
                for v in range(vregs_per_row):
                    col = v * _LANES
                    rows_v[r, pl.ds(col, _LANES)] = (
                        rows_v[r, pl.ds(col, _LANES)] + pe_v[r, pl.ds(col, _LANES)]
                    )
                return c2

            lax.fori_loop(0, _CHUNK, add_row, 0)
            pltpu.sync_copy(rows_v, out_hbm.at[pl.ds(tok0 + base, _CHUNK)])
            return carry

        lax.fori_loop(0, nsub, sub, 0)

    return k(W, idx_flat, pe)


def kernel(x, W):
    b, s = x.shape
    d_model = W.shape[1]
    pe = jnp.asarray(_pos_encoding(s, d_model))
    out = _embed(W, x.reshape(b * s), pe, s, d_model)
    return out.reshape(b, s, d_model)


# speedup vs baseline: 1.6507x; 1.6507x over previous
import functools

import jax
import jax.numpy as jnp
import numpy as np
from jax import lax
from jax.experimental import pallas as pl
from jax.experimental.pallas import tpu as pltpu
from jax.experimental.pallas import tpu_sc as plsc

_NC = 2
_NS = 16
_NW = _NC * _NS
_LANES = 16
_CHUNK = 16
_NBUF = 4


def _pos_encoding(max_len, d_model):
    pos = np.arange(max_len, dtype=np.float32)[:, None]
    i = np.arange(0, d_model, 2, dtype=np.float32)
    div = np.power(10000.0, i / d_model)
    pe = np.zeros((max_len, d_model), dtype=np.float32)
    pe[:, 0::2] = np.sin(pos / div)
    pe[:, 1::2] = np.cos(pos / div)
    return pe


@functools.partial(jax.jit, static_argnums=(3, 4))
def _embed(W, idx_flat, pe, seq, d_model):
    n = idx_flat.shape[0]
    per_w = n // _NW
    nsub = per_w // _CHUNK
    njj = nsub // _NBUF
    vregs_per_row = d_model // _LANES

    mesh = plsc.VectorSubcoreMesh(core_axis_name="c", subcore_axis_name="s")

    rows_t = pltpu.VMEM((_CHUNK, d_model), jnp.float32)
    pe_t = pltpu.VMEM((_CHUNK, d_model), jnp.float32)

    @functools.partial(
        pl.kernel,
        out_type=jax.ShapeDtypeStruct((n, d_model), jnp.float32),
        mesh=mesh,
        scratch_types=(
            [pltpu.VMEM((per_w,), jnp.int32)]
            + [rows_t] * _NBUF
            + [pe_t] * (_NBUF // 2)
            + [pltpu.SemaphoreType.DMA] * (2 * _NBUF + _NBUF // 2)
        ),
    )
    def k(w_hbm, idx_hbm, pe_hbm, out_hbm, idx_v, *bufs):
        rows = bufs[:_NBUF]
        pes = bufs[_NBUF : _NBUF + _NBUF // 2]
        gsem = bufs[_NBUF + _NBUF // 2 : 2 * _NBUF + _NBUF // 2]
        psem = bufs[2 * _NBUF + _NBUF // 2 : 2 * _NBUF + _NBUF]
        osem = bufs[2 * _NBUF + _NBUF :]

        wid = lax.axis_index("s") * _NC + lax.axis_index("c")
        tok0 = wid * per_w
        pos0 = lax.rem(tok0, seq)

        pltpu.sync_copy(idx_hbm.at[pl.ds(tok0, per_w)], idx_v)

        def gstart(c, k_):
            off = c * _CHUNK
            pltpu.async_copy(
                w_hbm.at[idx_v.at[pl.ds(off, _CHUNK)]], rows[k_], gsem[k_]
            )
            pltpu.async_copy(
                pe_hbm.at[pl.ds(pos0 + off, _CHUNK)],
                pes[k_ % (_NBUF // 2)],
                psem[k_ % (_NBUF // 2)],
            )

        def gwait(k_):
            pltpu.make_async_copy(
                pe_hbm.at[pl.ds(0, _CHUNK)], rows[k_], gsem[k_]
            ).wait()
            pltpu.make_async_copy(
                pe_hbm.at[pl.ds(0, _CHUNK)],
                pes[k_ % (_NBUF // 2)],
                psem[k_ % (_NBUF // 2)],
            ).wait()

        def ostart(c, k_):
            pltpu.async_copy(
                rows[k_], out_hbm.at[pl.ds(tok0 + c * _CHUNK, _CHUNK)], osem[k_]
            )

        def owait(k_):
            pltpu.make_async_copy(
                rows[k_], out_hbm.at[pl.ds(tok0, _CHUNK)], osem[k_]
            ).wait()

        def add_chunk(k_):
            pe_b = pes[k_ % (_NBUF // 2)]

            def add_row(r, carry):
                for v in range(vregs_per_row):
                    sl = pl.ds(v * _LANES, _LANES)
                    plsc.addupdate(rows[k_].at[r, sl], pe_b[r, sl])
                return carry

            lax.fori_loop(0, _CHUNK, add_row, 0)

        gstart(0, 0)
        gstart(1, 1)

        def body(jj, carry):
            for k_ in range(_NBUF):
                c = jj * _NBUF + k_
                k2 = (k_ + 2) % _NBUF
                gwait(k_)
                add_chunk(k_)
                ostart(c, k_)
                if k_ < 2:
                    @pl.when(jj > 0)
                    def _():
                        owait(k2)

                    gstart(c + 2, k2)
                else:
                    @pl.when(jj < njj - 1)
                    def _():
                        owait(k2)
                        gstart(c + 2, k2)

            return carry

        lax.fori_loop(0, njj, body, 0)

        for k_ in range(_NBUF):
            owait(k_)

    return k(W, idx_flat, pe)


def kernel(x, W):
    b, s = x.shape
    d_model = W.shape[1]
    pe = jnp.asarray(_pos_encoding(s, d_model))
    out = _embed(W, x.reshape(b * s), pe, s, d_model)
    return out.reshape(b, s, d_model)


# speedup vs baseline: 1.6728x; 1.0134x over previous
import functools

import jax
import jax.numpy as jnp
import numpy as np
from jax import lax
from jax.experimental import pallas as pl
from jax.experimental.pallas import tpu as pltpu
from jax.experimental.pallas import tpu_sc as plsc

_NC = 2
_NS = 16
_NW = _NC * _NS
_LANES = 16
_CHUNK = 8
_NBUF = 8
_PEB = 4
_DIST = 4


def _pos_encoding(max_len, d_model):
    pos = np.arange(max_len, dtype=np.float32)[:, None]
    i = np.arange(0, d_model, 2, dtype=np.float32)
    div = np.power(10000.0, i / d_model)
    pe = np.zeros((max_len, d_model), dtype=np.float32)
    pe[:, 0::2] = np.sin(pos / div)
    pe[:, 1::2] = np.cos(pos / div)
    return pe


@functools.partial(jax.jit, static_argnums=(3, 4))
def _embed(W, idx_flat, pe, seq, d_model):
    n = idx_flat.shape[0]
    per_w = n // _NW
    nsub = per_w // _CHUNK
    njj = nsub // _NBUF
    vregs_per_row = d_model // _LANES

    mesh = plsc.VectorSubcoreMesh(core_axis_name="c", subcore_axis_name="s")

    rows_t = pltpu.VMEM((_CHUNK, d_model), jnp.float32)
    pe_t = pltpu.VMEM((_CHUNK, d_model), jnp.float32)

    @functools.partial(
        pl.kernel,
        out_type=jax.ShapeDtypeStruct((n, d_model), jnp.float32),
        mesh=mesh,
        scratch_types=(
            [pltpu.VMEM((per_w,), jnp.int32)]
            + [rows_t] * _NBUF
            + [pe_t] * _PEB
            + [pltpu.SemaphoreType.DMA] * (2 * _NBUF + _PEB)
        ),
    )
    def k(w_hbm, idx_hbm, pe_hbm, out_hbm, idx_v, *bufs):
        rows = bufs[:_NBUF]
        pes = bufs[_NBUF : _NBUF + _PEB]
        gsem = bufs[_NBUF + _PEB : 2 * _NBUF + _PEB]
        psem = bufs[2 * _NBUF + _PEB : 2 * _NBUF + 2 * _PEB]
        osem = bufs[2 * _NBUF + 2 * _PEB :]

        wid = lax.axis_index("s") * _NC + lax.axis_index("c")
        tok0 = wid * per_w
        pos0 = lax.rem(tok0, seq)

        pltpu.sync_copy(idx_hbm.at[pl.ds(tok0, per_w)], idx_v)

        def gstart(c, k_):
            pltpu.async_copy(
                w_hbm.at[idx_v.at[pl.ds(c * _CHUNK, _CHUNK)]], rows[k_], gsem[k_]
            )

        def pstart(c, k_):
            pltpu.async_copy(
                pe_hbm.at[pl.ds(pos0 + c * _CHUNK, _CHUNK)],
                pes[k_ % _PEB],
                psem[k_ % _PEB],
            )

        def gwait(k_):
            pltpu.make_async_copy(
                pe_hbm.at[pl.ds(0, _CHUNK)], rows[k_], gsem[k_]
            ).wait()
            pltpu.make_async_copy(
                pe_hbm.at[pl.ds(0, _CHUNK)], pes[k_ % _PEB], psem[k_ % _PEB]
            ).wait()

        def ostart(c, k_):
            pltpu.async_copy(
                rows[k_], out_hbm.at[pl.ds(tok0 + c * _CHUNK, _CHUNK)], osem[k_]
            )

        def owait(k_):
            pltpu.make_async_copy(
                rows[k_], out_hbm.at[pl.ds(tok0, _CHUNK)], osem[k_]
            ).wait()

        def add_chunk(k_):
            pe_b = pes[k_ % _PEB]

            def add_row(r, carry):
                for v in range(vregs_per_row):
                    sl = pl.ds(v * _LANES, _LANES)
                    plsc.addupdate(rows[k_].at[r, sl], pe_b[r, sl])
                return carry

            lax.fori_loop(0, _CHUNK, add_row, 0)

        for c in range(_DIST):
            gstart(c, c)
            pstart(c, c)

        def body(jj, carry):
            for k_ in range(_NBUF):
                c = jj * _NBUF + k_
                kd = (k_ + _DIST) % _NBUF
                if k_ < _DIST:
                    @pl.when(jj > 0)
                    def _():
                        owait(kd)

                    gstart(c + _DIST, kd)
                    gwait(k_)
                    add_chunk(k_)
                    pstart(c + _DIST, kd)
                else:
                    @pl.when(jj < njj - 1)
                    def _():
                        owait(kd)
                        gstart(c + _DIST, kd)

                    gwait(k_)
                    add_chunk(k_)

                    @pl.when(jj < njj - 1)
                    def _():
                        pstart(c + _DIST, kd)

                ostart(c, k_)

            return carry

        lax.fori_loop(0, njj, body, 0)

        for k_ in range(_NBUF):
            owait(k_)

    return k(W, idx_flat, pe)


def kernel(x, W):
    b, s = x.shape
    d_model = W.shape[1]
    pe = jnp.asarray(_pos_encoding(s, d_model))
    out = _embed(W, x.reshape(b * s), pe, s, d_model)
    return out.reshape(b, s, d_model)


# speedup vs baseline: 1.9461x; 1.1634x over previous
import functools

import jax
import jax.numpy as jnp
import numpy as np
from jax import lax
from jax.experimental import pallas as pl
from jax.experimental.pallas import tpu as pltpu
from jax.experimental.pallas import tpu_sc as plsc

_NC = 2
_NS = 16
_NW = _NC * _NS
_LANES = 16
_CHUNK = 8
_NBUF = 8
_PEB = 2
_DIST = 4


def _pos_encoding(max_len, d_model):
    pos = np.arange(max_len, dtype=np.float32)[:, None]
    i = np.arange(0, d_model, 2, dtype=np.float32)
    div = np.power(10000.0, i / d_model)
    pe = np.zeros((max_len, d_model), dtype=np.float32)
    pe[:, 0::2] = np.sin(pos / div)
    pe[:, 1::2] = np.cos(pos / div)
    return pe


@functools.partial(jax.jit, static_argnums=(3, 4, 5))
def _embed(W, idx_flat, pe, batch, seq, d_model):
    n = idx_flat.shape[0]
    per_w = n // _NW
    posn = seq // _NW
    nsub = per_w // _CHUNK
    njj = nsub // _NBUF
    vregs_per_row = d_model // _LANES

    mesh = plsc.VectorSubcoreMesh(core_axis_name="c", subcore_axis_name="s")

    rows_t = pltpu.VMEM((_CHUNK, d_model), jnp.float32)
    pe_t = pltpu.VMEM((_CHUNK, d_model), jnp.float32)

    @functools.partial(
        pl.kernel,
        out_type=jax.ShapeDtypeStruct((n, d_model), jnp.float32),
        mesh=mesh,
        scratch_types=(
            [pltpu.VMEM((per_w,), jnp.int32)]
            + [rows_t] * _NBUF
            + [pe_t] * _PEB
            + [pltpu.SemaphoreType.DMA] * (2 * _NBUF + _PEB)
        ),
    )
    def k(w_hbm, idx_hbm, pe_hbm, out_hbm, idx_v, *bufs):
        rows = bufs[:_NBUF]
        pes = bufs[_NBUF : _NBUF + _PEB]
        gsem = bufs[_NBUF + _PEB : 2 * _NBUF + _PEB]
        psem = bufs[2 * _NBUF + _PEB : 2 * _NBUF + 2 * _PEB]
        osem = bufs[2 * _NBUF + 2 * _PEB :]

        wid = lax.axis_index("s") * _NC + lax.axis_index("c")
        pos0 = wid * posn

        for b in range(batch):
            pltpu.sync_copy(
                idx_hbm.at[pl.ds(b * seq + pos0, posn)],
                idx_v.at[pl.ds(b * posn, posn)],
            )

        def gstart(c, b, k_):
            pltpu.async_copy(
                w_hbm.at[
                    idx_v.at[
                        pl.ds(b * posn + (c // batch) * _CHUNK, _CHUNK)
                    ]
                ],
                rows[k_],
                gsem[k_],
            )

        def pstart(pc, pb):
            pltpu.async_copy(
                pe_hbm.at[pl.ds(pos0 + pc * _CHUNK, _CHUNK)], pes[pb], psem[pb]
            )

        def gwait(k_):
            pltpu.make_async_copy(
                pe_hbm.at[pl.ds(0, _CHUNK)], rows[k_], gsem[k_]
            ).wait()

        def pwait(pb):
            pltpu.make_async_copy(
                pe_hbm.at[pl.ds(0, _CHUNK)], pes[pb], psem[pb]
            ).wait()

        def ostart(c, b, k_):
            pltpu.async_copy(
                rows[k_],
                out_hbm.at[
                    pl.ds(b * seq + pos0 + (c // batch) * _CHUNK, _CHUNK)
                ],
                osem[k_],
            )

        def owait(k_):
            pltpu.make_async_copy(
                rows[k_], out_hbm.at[pl.ds(0, _CHUNK)], osem[k_]
            ).wait()

        def add_chunk(k_, pb):
            pe_b = pes[pb]

            def add_row(r, carry):
                for v in range(vregs_per_row):
                    sl = pl.ds(v * _LANES, _LANES)
                    plsc.addupdate(rows[k_].at[r, sl], pe_b[r, sl])
                return carry

            lax.fori_loop(0, _CHUNK, add_row, 0)

        pstart(0, 0)
        for c in range(_DIST):
            gstart(c, c % batch, c)

        def body(jj, carry):
            for k_ in range(_NBUF):
                c = jj * _NBUF + k_
                b = k_ % batch
                kd = (k_ + _DIST) % _NBUF
                bd = (k_ + _DIST) % batch
                pb = (k_ // batch) % _PEB
                pbd = (k_ // batch + 1) % _PEB

                have_next = (
                    (jj < njj - 1) if k_ + _DIST >= _NBUF else (jj >= 0)
                )

                @pl.when(have_next if k_ + _DIST >= _NBUF else jj > 0)
                def _():
                    owait(kd)

                @pl.when(have_next)
                def _():
                    gstart(c + _DIST, bd, kd)

                if (k_ + _DIST) % batch == 0:
                    @pl.when(have_next)
                    def _():
                        pstart((c + _DIST) // batch, pbd)

                gwait(k_)
                if k_ % batch == 0:
                    pwait(pb)
                add_chunk(k_, pb)
                ostart(c, b, k_)

            return carry

        lax.fori_loop(0, njj, body, 0)

        for k_ in range(_NBUF):
            owait(k_)

    return k(W, idx_flat, pe)


def kernel(x, W):
    b, s = x.shape
    d_model = W.shape[1]
    pe = jnp.asarray(_pos_encoding(s, d_model))
    out = _embed(W, x.reshape(b * s), pe, b, s, d_model)
    return out.reshape(b, s, d_model)


# speedup vs baseline: 1.9524x; 1.0032x over previous
import functools

import jax
import jax.numpy as jnp
import numpy as np
from jax import lax
from jax.experimental import pallas as pl
from jax.experimental.pallas import tpu as pltpu
from jax.experimental.pallas import tpu_sc as plsc

_NC = 2
_NS = 16
_NW = _NC * _NS
_LANES = 16
_CHUNK = 8
_NBUF = 8
_PEB = 2
_DIST = 5


def _pos_encoding(max_len, d_model):
    pos = np.arange(max_len, dtype=np.float32)[:, None]
    i = np.arange(0, d_model, 2, dtype=np.float32)
    div = np.power(10000.0, i / d_model)
    pe = np.zeros((max_len, d_model), dtype=np.float32)
    pe[:, 0::2] = np.sin(pos / div)
    pe[:, 1::2] = np.cos(pos / div)
    return pe


@functools.partial(jax.jit, static_argnums=(3, 4, 5))
def _embed(W, idx_flat, pe, batch, seq, d_model):
    n = idx_flat.shape[0]
    per_w = n // _NW
    posn = seq // _NW
    nsub = per_w // _CHUNK
    njj = nsub // _NBUF
    vregs_per_row = d_model // _LANES

    mesh = plsc.VectorSubcoreMesh(core_axis_name="c", subcore_axis_name="s")

    rows_t = pltpu.VMEM((_CHUNK, d_model), jnp.float32)
    pe_t = pltpu.VMEM((_CHUNK, d_model), jnp.float32)

    @functools.partial(
        pl.kernel,
        out_type=jax.ShapeDtypeStruct((n, d_model), jnp.float32),
        mesh=mesh,
        scratch_types=(
            [pltpu.VMEM((per_w,), jnp.int32)]
            + [rows_t] * _NBUF
            + [pe_t] * _PEB
            + [pltpu.SemaphoreType.DMA] * (2 * _NBUF + _PEB)
        ),
    )
    def k(w_hbm, idx_hbm, pe_hbm, out_hbm, idx_v, *bufs):
        rows = bufs[:_NBUF]
        pes = bufs[_NBUF : _NBUF + _PEB]
        gsem = bufs[_NBUF + _PEB : 2 * _NBUF + _PEB]
        psem = bufs[2 * _NBUF + _PEB : 2 * _NBUF + 2 * _PEB]
        osem = bufs[2 * _NBUF + 2 * _PEB :]

        wid = lax.axis_index("s") * _NC + lax.axis_index("c")
        pos0 = wid * posn

        for b in range(batch):
            pltpu.sync_copy(
                idx_hbm.at[pl.ds(b * seq + pos0, posn)],
                idx_v.at[pl.ds(b * posn, posn)],
            )

        def gstart(c, b, k_):
            pltpu.async_copy(
                w_hbm.at[
                    idx_v.at[
                        pl.ds(b * posn + (c // batch) * _CHUNK, _CHUNK)
                    ]
                ],
                rows[k_],
                gsem[k_],
            )

        def pstart(pc, pb):
            pltpu.async_copy(
                pe_hbm.at[pl.ds(pos0 + pc * _CHUNK, _CHUNK)], pes[pb], psem[pb]
            )

        def gwait(k_):
            pltpu.make_async_copy(
                pe_hbm.at[pl.ds(0, _CHUNK)], rows[k_], gsem[k_]
            ).wait()

        def pwait(pb):
            pltpu.make_async_copy(
                pe_hbm.at[pl.ds(0, _CHUNK)], pes[pb], psem[pb]
            ).wait()

        def ostart(c, b, k_):
            pltpu.async_copy(
                rows[k_],
                out_hbm.at[
                    pl.ds(b * seq + pos0 + (c // batch) * _CHUNK, _CHUNK)
                ],
                osem[k_],
            )

        def owait(k_):
            pltpu.make_async_copy(
                rows[k_], out_hbm.at[pl.ds(0, _CHUNK)], osem[k_]
            ).wait()

        def add_chunk(k_, pb):
            pe_b = pes[pb]

            def add_row(r, carry):
                for v in range(vregs_per_row):
                    sl = pl.ds(v * _LANES, _LANES)
                    plsc.addupdate(rows[k_].at[r, sl], pe_b[r, sl])
                return carry

            lax.fori_loop(0, _CHUNK, add_row, 0)

        pstart(0, 0)
        for c in range(_DIST):
            gstart(c, c % batch, c)

        def body(jj, carry):
            for k_ in range(_NBUF):
                c = jj * _NBUF + k_
                b = k_ % batch
                kd = (k_ + _DIST) % _NBUF
                bd = (k_ + _DIST) % batch
                pb = (k_ // batch) % _PEB

                have_next = True if k_ + _DIST < _NBUF else (jj < njj - 1)
                have_prev = True if k_ + _DIST >= _NBUF else (jj > 0)

                if k_ + _DIST < _NBUF:
                    @pl.when(have_prev)
                    def _():
                        owait(kd)

                    gstart(c + _DIST, bd, kd)
                else:
                    @pl.when(have_next)
                    def _():
                        owait(kd)
                        gstart(c + _DIST, bd, kd)

                gwait(k_)
                if k_ % batch == 0:
                    pwait(pb)
                    pc_next_exists = (
                        True if k_ + batch < _NBUF else (jj < njj - 1)
                    )

                    @pl.when(pc_next_exists)
                    def _():
                        pstart(c // batch + 1, (k_ // batch + 1) % _PEB)

                add_chunk(k_, pb)
                ostart(c, b, k_)

            return carry

        lax.fori_loop(0, njj, body, 0)

        for k_ in range(_NBUF):
            owait(k_)

    return k(W, idx_flat, pe)


def kernel(x, W):
    b, s = x.shape
    d_model = W.shape[1]
    pe = jnp.asarray(_pos_encoding(s, d_model))
    out = _embed(W, x.reshape(b * s), pe, b, s, d_model)
    return out.reshape(b, s, d_model)


# speedup vs baseline: 1.9699x; 1.0090x over previous
import functools

import jax
import jax.numpy as jnp
import numpy as np
from jax import lax
from jax.experimental import pallas as pl
from jax.experimental.pallas import tpu as pltpu
from jax.experimental.pallas import tpu_sc as plsc

_NC = 2
_NS = 16
_NW = _NC * _NS
_LANES = 16
_CHUNK = 16
_NBUF = 4
_SLOTS = 8
_PEB = 2
_DIST = 2


def _pos_encoding(max_len, d_model):
    pos = np.arange(max_len, dtype=np.float32)[:, None]
    i = np.arange(0, d_model, 2, dtype=np.float32)
    div = np.power(10000.0, i / d_model)
    pe = np.zeros((max_len, d_model), dtype=np.float32)
    pe[:, 0::2] = np.sin(pos / div)
    pe[:, 1::2] = np.cos(pos / div)
    return pe


@functools.partial(jax.jit, static_argnums=(3, 4, 5))
def _embed(W, idx_flat, pe, batch, seq, d_model):
    n = idx_flat.shape[0]
    per_w = n // _NW
    posn = seq // _NW
    nsub = per_w // _CHUNK
    njj = nsub // _SLOTS
    vregs_per_row = d_model // _LANES

    mesh = plsc.VectorSubcoreMesh(core_axis_name="c", subcore_axis_name="s")

    rows_t = pltpu.VMEM((_CHUNK, d_model), jnp.float32)
    pe_t = pltpu.VMEM((_CHUNK, d_model), jnp.float32)

    @functools.partial(
        pl.kernel,
        out_type=jax.ShapeDtypeStruct((n, d_model), jnp.float32),
        mesh=mesh,
        scratch_types=(
            [pltpu.VMEM((per_w,), jnp.int32)]
            + [rows_t] * _NBUF
            + [pe_t] * _PEB
            + [pltpu.SemaphoreType.DMA] * (2 * _NBUF + _PEB)
        ),
    )
    def k(w_hbm, idx_hbm, pe_hbm, out_hbm, idx_v, *bufs):
        rows = bufs[:_NBUF]
        pes = bufs[_NBUF : _NBUF + _PEB]
        gsem = bufs[_NBUF + _PEB : 2 * _NBUF + _PEB]
        psem = bufs[2 * _NBUF + _PEB : 2 * _NBUF + 2 * _PEB]
        osem = bufs[2 * _NBUF + 2 * _PEB :]

        wid = lax.axis_index("s") * _NC + lax.axis_index("c")
        pos0 = wid * posn

        idx_copies = [
            pltpu.async_copy(
                idx_hbm.at[pl.ds(b * seq + pos0, posn)],
                idx_v.at[pl.ds(b * posn, posn)],
                gsem[0],
            )
            for b in range(batch)
        ]
        for cp in idx_copies:
            cp.wait()

        def gstart(c, b, k_):
            pltpu.async_copy(
                w_hbm.at[
                    idx_v.at[
                        pl.ds(b * posn + (c // batch) * _CHUNK, _CHUNK)
                    ]
                ],
                rows[k_],
                gsem[k_],
            )

        def pstart(pc, pb):
            pltpu.async_copy(
                pe_hbm.at[pl.ds(pos0 + pc * _CHUNK, _CHUNK)], pes[pb], psem[pb]
            )

        def gwait(k_):
            pltpu.make_async_copy(
                pe_hbm.at[pl.ds(0, _CHUNK)], rows[k_], gsem[k_]
            ).wait()

        def pwait(pb):
            pltpu.make_async_copy(
                pe_hbm.at[pl.ds(0, _CHUNK)], pes[pb], psem[pb]
            ).wait()

        def ostart(c, b, k_):
            pltpu.async_copy(
                rows[k_],
                out_hbm.at[
                    pl.ds(b * seq + pos0 + (c // batch) * _CHUNK, _CHUNK)
                ],
                osem[k_],
            )

        def owait(k_):
            pltpu.make_async_copy(
                rows[k_], out_hbm.at[pl.ds(0, _CHUNK)], osem[k_]
            ).wait()

        def add_chunk(k_, pb):
            pe_b = pes[pb]

            def add_row(r, carry):
                for v in range(vregs_per_row):
                    sl = pl.ds(v * _LANES, _LANES)
                    plsc.addupdate(rows[k_].at[r, sl], pe_b[r, sl])
                return carry

            lax.fori_loop(0, _CHUNK, add_row, 0)

        pstart(0, 0)
        for c in range(_DIST):
            gstart(c, c % batch, c)

        def body(jj, carry):
            for k_ in range(_SLOTS):
                c = jj * _SLOTS + k_
                kb = k_ % _NBUF
                b = k_ % batch
                kd = (k_ + _DIST) % _NBUF
                bd = (k_ + _DIST) % batch
                pb = (k_ // batch) % _PEB

                have_next = True if k_ + _DIST < _SLOTS else (jj < njj - 1)
                have_prev = True if k_ + _DIST >= _NBUF else (jj > 0)

                if k_ + _DIST < _SLOTS:
                    @pl.when(have_prev)
                    def _():
                        owait(kd)

                    gstart(c + _DIST, bd, kd)
                else:
                    @pl.when(have_next)
                    def _():
                        owait(kd)
                        gstart(c + _DIST, bd, kd)

                gwait(kb)
                if k_ % batch == 0:
                    pwait(pb)
                    pc_next_exists = (
                        True if k_ + batch < _SLOTS else (jj < njj - 1)
                    )

                    @pl.when(pc_next_exists)
                    def _():
                        pstart(c // batch + 1, (k_ // batch + 1) % _PEB)

                add_chunk(kb, pb)
                ostart(c, b, kb)

            return carry

        lax.fori_loop(0, njj, body, 0)

        for k_ in range(_NBUF):
            owait(k_)

    return k(W, idx_flat, pe)


def kernel(x, W):
    b, s = x.shape
    d_model = W.shape[1]
    pe = jnp.asarray(_pos_encoding(s, d_model))
    out = _embed(W, x.reshape(b * s), pe, b, s, d_model)
    return out.reshape(b, s, d_model)
